# baseline (speedup 1.0000x reference)
import jax
import jax.numpy as jnp
from jax import lax
from jax.experimental import pallas as pl
import jax.experimental.pallas.tpu as pltpu
from jax.experimental.pallas import tpu_sc as plsc

_B = 16
_TB = 10
_SB = _B - _TB
_C = 768
_P = 512
_HW = 1024
_D = _C + _P

_K = 8
_W = 4


def _tc_body(x_hbm, row_ref, col_ref, o_hbm, stage, in_sems, out_sems):
    colb = jnp.broadcast_to(col_ref[...][None, :, :], (32, 32, 256)).reshape(_HW, 256)
    rowb = jnp.broadcast_to(row_ref[...][:, None, :], (32, 32, 256)).reshape(_HW, 256)
    for s in range(_K):
        stage[s, :, _C:_C + 256] = colb
        stage[s, :, _C + 256:] = rowb

    def in_copy(i):
        return pltpu.make_async_copy(
            x_hbm.at[i], stage.at[i % _K, :, 0:_C], in_sems.at[i % _K])

    out_copies = [
        pltpu.make_async_copy(stage.at[i % _K], o_hbm.at[i], out_sems.at[i % _K])
        for i in range(_TB)
    ]
    for i in range(min(_K, _TB)):
        in_copy(i).start()
    for i in range(_TB):
        in_copy(i).wait()
        out_copies[i].start()
        j = i - _W
        if j >= 0 and j + _K < _TB:
            out_copies[j].wait()
            in_copy(j + _K).start()
    for i in range(max(0, _TB - _K), _TB):
        out_copies[i].wait()


_CR = 16
_ROWS_W = _SB * _HW // 32
_NCH = _ROWS_W // _CR
_SK = 4
_SW = 2


def _sc_body(x_hbm, row_hbm, col_hbm, o_hbm, colbuf, rowbuf, stage,
             in_sems, out_sems):
    cid = lax.axis_index("c")
    sid = lax.axis_index("s")
    wid = sid * 2 + cid

    pltpu.sync_copy(col_hbm, colbuf)
    pltpu.sync_copy(row_hbm, rowbuf)

    for k in range(_SK):
        wlo = (k % 2) * _CR
        for r in range(_CR):
            for g in range(16):
                stage[k, r, pl.ds(_C + g * 16, 16)] = colbuf[wlo + r, pl.ds(g * 16, 16)]

    def chunk_addr(i):
        f = wid * _ROWS_W + i * _CR
        return f // _HW, lax.rem(f, _HW)

    def in_copy(i, k):
        b, p = chunk_addr(i)
        return pltpu.make_async_copy(
            x_hbm.at[_TB + b, pl.ds(p, _CR)],
            stage.at[k, :, pl.ds(0, _C)], in_sems.at[k])

    def out_copy(i, k):
        b, p = chunk_addr(i)
        return pltpu.make_async_copy(
            stage.at[k],
            o_hbm.at[b, pl.ds(p, _CR)], out_sems.at[k])

    for i in range(_SK):
        in_copy(i, i % _SK).start()

    def step(i, carry):
        k = lax.rem(i, _SK)
        in_copy(i, k).wait()
        _, p = chunk_addr(i)
        h = p // 32
        for g in range(16):
            v = rowbuf[h, pl.ds(g * 16, 16)]
            for r in range(_CR):
                stage[k, r, pl.ds(_C + 256 + g * 16, 16)] = v
        out_copy(i, k).start()
        j = i - _SW

        @pl.when(jnp.logical_and(j >= 0, j + _SK < _NCH))
        def _():
            out_copy(j, lax.rem(j, _SK)).wait()
            in_copy(j + _SK, lax.rem(j, _SK)).start()

        return carry

    lax.fori_loop(0, _NCH, step, 0)
    for i in range(_NCH - _SK, _NCH):
        out_copy(i, i % _SK).wait()


def kernel(x, row_embed, col_embed):
    bsz, c, h, w = x.shape
    xt = x.transpose(0, 2, 3, 1).reshape(bsz, h * w, c)

    sc_run = pl.kernel(
        _sc_body,
        out_type=jax.ShapeDtypeStruct((_SB, h * w, _D), x.dtype),
        mesh=plsc.VectorSubcoreMesh(core_axis_name="c", subcore_axis_name="s"),
        scratch_types=[
            pltpu.VMEM((32, 256), x.dtype),
            pltpu.VMEM((32, 256), x.dtype),
            pltpu.VMEM((_SK, _CR, _D), x.dtype),
            pltpu.SemaphoreType.DMA((_SK,)),
            pltpu.SemaphoreType.DMA((_SK,)),
        ],
    )
    sc_out = sc_run(xt, row_embed, col_embed)

    tc_out = pl.pallas_call(
        _tc_body,
        in_specs=[
            pl.BlockSpec(memory_space=pl.ANY),
            pl.BlockSpec(memory_space=pltpu.MemorySpace.VMEM),
            pl.BlockSpec(memory_space=pltpu.MemorySpace.VMEM),
        ],
        out_specs=pl.BlockSpec(memory_space=pl.ANY),
        out_shape=jax.ShapeDtypeStruct((_TB, h * w, _D), x.dtype),
        scratch_shapes=[
            pltpu.VMEM((_K, h * w, _D), x.dtype),
            pltpu.SemaphoreType.DMA((_K,)),
            pltpu.SemaphoreType.DMA((_K,)),
        ],
    )(xt, row_embed, col_embed)

    out = jnp.concatenate([tc_out, sc_out], axis=0)
    return out.reshape(bsz, h, w, _D).transpose(0, 3, 1, 2)


# speedup vs baseline: 2.9014x; 2.9014x over previous
import jax
import jax.numpy as jnp
from jax.experimental import pallas as pl
import jax.experimental.pallas.tpu as pltpu

_B = 16
_C = 768
_P = 512
_HW = 1024
_KI = 4
_KO = 6


def _concat_pos_kernel(x_hbm, row_ref, col_ref, o_hbm, cbuf, tile, in_sems, out_sems):
    colb = jnp.broadcast_to(col_ref[...][None, :, :], (32, 32, 256)).reshape(_HW, 256)
    rowb = jnp.broadcast_to(row_ref[...][:, None, :], (32, 32, 256)).reshape(_HW, 256)
    for s in range(_KO):
        tile[s, :, _C:_C + 256] = colb
        tile[s, :, _C + 256:] = rowb

    def in_copy(i):
        return pltpu.make_async_copy(x_hbm.at[i], cbuf.at[i % _KI], in_sems.at[i % _KI])

    out_copies = [
        pltpu.make_async_copy(tile.at[i % _KO], o_hbm.at[i], out_sems.at[i % _KO])
        for i in range(_B)
    ]

    for i in range(_KI):
        in_copy(i).start()
    for i in range(_B):
        in_copy(i).wait()
        if i >= _KO:
            out_copies[i - _KO].wait()
        tile[i % _KO, :, 0:_C] = cbuf[i % _KI]
        out_copies[i].start()
        if i + _KI < _B:
            in_copy(i + _KI).start()
    for i in range(_B - _KO, _B):
        out_copies[i].wait()


def kernel(x, row_embed, col_embed):
    b, c, h, w = x.shape
    xt = x.transpose(0, 2, 3, 1).reshape(b, h * w, c)
    out = pl.pallas_call(
        _concat_pos_kernel,
        in_specs=[
            pl.BlockSpec(memory_space=pl.ANY),
            pl.BlockSpec(memory_space=pltpu.MemorySpace.VMEM),
            pl.BlockSpec(memory_space=pltpu.MemorySpace.VMEM),
        ],
        out_specs=pl.BlockSpec(memory_space=pl.ANY),
        out_shape=jax.ShapeDtypeStruct((b, h * w, c + _P), x.dtype),
        scratch_shapes=[
            pltpu.VMEM((_KI, h * w, c), x.dtype),
            pltpu.VMEM((_KO, h * w, c + _P), x.dtype),
            pltpu.SemaphoreType.DMA((_KI,)),
            pltpu.SemaphoreType.DMA((_KO,)),
        ],
    )(xt, row_embed, col_embed)
    return out.reshape(b, h, w, c + _P).transpose(0, 3, 1, 2)
